# Initial kernel scaffold; baseline (speedup 1.0000x reference)
import jax
import jax.numpy as jnp
from jax.experimental import pallas as pl


def kernel(features, Wg, bg, W1, b1, W2, b2):
    raise NotImplementedError("write your pallas kernel here")


# speedup vs baseline: 1.2043x; 1.2043x over previous
import functools

import jax
import jax.numpy as jnp
from jax.experimental import pallas as pl

_TB = 512


def _moe_block_kernel(x_ref, wg_ref, bg_ref, w1_ref, b1_ref, w2_ref, b2_ref,
                      out_ref, *, n_experts, d_hidden):
    x = x_ref[...]
    logits = jnp.dot(x, wg_ref[...], preferred_element_type=jnp.float32)
    logits = logits + bg_ref[...]
    gates = jax.nn.softmax(logits, axis=-1)

    h = jnp.dot(x, w1_ref[...], preferred_element_type=jnp.float32)
    h = jnp.maximum(h + b1_ref[...], 0.0)

    tb = h.shape[0]
    hg = (h.reshape(tb, n_experts, d_hidden) * gates[:, :, None]
          ).reshape(tb, n_experts * d_hidden)

    out = jnp.dot(hg, w2_ref[...], preferred_element_type=jnp.float32)
    out = out + jnp.dot(gates, b2_ref[...],
                        preferred_element_type=jnp.float32)
    out_ref[...] = out


def kernel(features, Wg, bg, W1, b1, W2, b2):
    t, d = features.shape
    e, _, h = W1.shape
    a = W2.shape[-1]

    w1c = W1.transpose(1, 0, 2).reshape(d, e * h)
    b1c = b1.reshape(1, e * h)
    w2c = W2.reshape(e * h, a)
    bg2 = bg.reshape(1, e)

    grid = (t // _TB,)
    body = functools.partial(_moe_block_kernel, n_experts=e, d_hidden=h)
    return pl.pallas_call(
        body,
        grid=grid,
        in_specs=[
            pl.BlockSpec((_TB, d), lambda i: (i, 0)),
            pl.BlockSpec((d, e), lambda i: (0, 0)),
            pl.BlockSpec((1, e), lambda i: (0, 0)),
            pl.BlockSpec((d, e * h), lambda i: (0, 0)),
            pl.BlockSpec((1, e * h), lambda i: (0, 0)),
            pl.BlockSpec((e * h, a), lambda i: (0, 0)),
            pl.BlockSpec((e, a), lambda i: (0, 0)),
        ],
        out_specs=pl.BlockSpec((_TB, a), lambda i: (i, 0)),
        out_shape=jax.ShapeDtypeStruct((t, a), jnp.float32),
    )(features, Wg, bg2, w1c, b1c, w2c, b2)


# speedup vs baseline: 1.3607x; 1.1299x over previous
import functools

import jax
import jax.numpy as jnp
from jax.experimental import pallas as pl

_TB = 512


def _moe_block_kernel(x_ref, wg_ref, bg_ref, w1_ref, b1_ref, w2_ref, b2_ref,
                      s_ref, out_ref):
    x = x_ref[...]
    logits = jnp.dot(x, wg_ref[...], preferred_element_type=jnp.float32)
    logits = logits + bg_ref[...]
    gates = jax.nn.softmax(logits, axis=-1)
    gate_full = jnp.dot(gates.astype(jnp.bfloat16), s_ref[...],
                        preferred_element_type=jnp.float32)

    h = jnp.dot(x, w1_ref[...], preferred_element_type=jnp.float32)
    hg = jnp.maximum(h + b1_ref[...], 0.0) * gate_full

    out = jnp.dot(hg.astype(jnp.bfloat16), w2_ref[...],
                  preferred_element_type=jnp.float32)
    out = out + jnp.dot(gates, b2_ref[...],
                        preferred_element_type=jnp.float32)
    out_ref[...] = out


def kernel(features, Wg, bg, W1, b1, W2, b2):
    t, d = features.shape
    e, _, h = W1.shape
    a = W2.shape[-1]

    xb = features.astype(jnp.bfloat16)
    wgb = Wg.astype(jnp.bfloat16)
    w1c = W1.transpose(1, 0, 2).reshape(d, e * h).astype(jnp.bfloat16)
    b1c = b1.reshape(1, e * h)
    w2c = W2.reshape(e * h, a).astype(jnp.bfloat16)
    bg2 = bg.reshape(1, e)
    s = jnp.kron(jnp.eye(e, dtype=jnp.bfloat16),
                 jnp.ones((1, h), dtype=jnp.bfloat16))

    grid = (t // _TB,)
    return pl.pallas_call(
        _moe_block_kernel,
        grid=grid,
        in_specs=[
            pl.BlockSpec((_TB, d), lambda i: (i, 0)),
            pl.BlockSpec((d, e), lambda i: (0, 0)),
            pl.BlockSpec((1, e), lambda i: (0, 0)),
            pl.BlockSpec((d, e * h), lambda i: (0, 0)),
            pl.BlockSpec((1, e * h), lambda i: (0, 0)),
            pl.BlockSpec((e * h, a), lambda i: (0, 0)),
            pl.BlockSpec((e, a), lambda i: (0, 0)),
            pl.BlockSpec((e, e * h), lambda i: (0, 0)),
        ],
        out_specs=pl.BlockSpec((_TB, a), lambda i: (i, 0)),
        out_shape=jax.ShapeDtypeStruct((t, a), jnp.float32),
    )(xb, wgb, bg2, w1c, b1c, w2c, b2, s)


# speedup vs baseline: 1.4815x; 1.0887x over previous
import numpy as np

import jax
import jax.numpy as jnp
from jax.experimental import pallas as pl

_TB = 512


def _moe_block_kernel(x_ref, wg_ref, bg_ref, w1_ref, b1_ref, w2_ref, b2_ref,
                      s_ref, out_ref):
    n_experts, _, d_hidden = w1_ref.shape
    x = x_ref[...].astype(jnp.bfloat16)

    logits = jnp.dot(x, wg_ref[...], preferred_element_type=jnp.float32)
    logits = logits + bg_ref[...]
    expg = jnp.exp(logits - jnp.max(logits, axis=-1, keepdims=True))
    denom = jnp.sum(expg, axis=-1, keepdims=True)
    expg16 = expg.astype(jnp.bfloat16)
    gate_full = jnp.dot(expg16, s_ref[...],
                        preferred_element_type=jnp.float32)

    acc = jnp.dot(expg16, b2_ref[...], preferred_element_type=jnp.float32)
    for e in range(n_experts):
        h = jnp.dot(x, w1_ref[e], preferred_element_type=jnp.float32)
        h = jnp.maximum(h + b1_ref[e:e + 1, :], 0.0)
        hg = h * gate_full[:, e * d_hidden:(e + 1) * d_hidden]
        acc = acc + jnp.dot(hg.astype(jnp.bfloat16), w2_ref[e],
                            preferred_element_type=jnp.float32)

    out_ref[...] = acc / denom


def kernel(features, Wg, bg, W1, b1, W2, b2):
    t, d = features.shape
    e, _, h = W1.shape
    a = W2.shape[-1]

    wgb = Wg.astype(jnp.bfloat16)
    w1b = W1.astype(jnp.bfloat16)
    w2b = W2.astype(jnp.bfloat16)
    bg2 = bg.reshape(1, e)
    s = jnp.asarray(np.kron(np.eye(e, dtype=np.float32),
                            np.ones((1, h), np.float32)), dtype=jnp.bfloat16)

    grid = (t // _TB,)
    return pl.pallas_call(
        _moe_block_kernel,
        grid=grid,
        in_specs=[
            pl.BlockSpec((_TB, d), lambda i: (i, 0)),
            pl.BlockSpec((d, e), lambda i: (0, 0)),
            pl.BlockSpec((1, e), lambda i: (0, 0)),
            pl.BlockSpec((e, d, h), lambda i: (0, 0, 0)),
            pl.BlockSpec((e, h), lambda i: (0, 0)),
            pl.BlockSpec((e, h, a), lambda i: (0, 0, 0)),
            pl.BlockSpec((e, a), lambda i: (0, 0)),
            pl.BlockSpec((e, e * h), lambda i: (0, 0)),
        ],
        out_specs=pl.BlockSpec((_TB, a), lambda i: (i, 0)),
        out_shape=jax.ShapeDtypeStruct((t, a), jnp.float32),
    )(features, wgb, bg2, w1b, b1, w2b, b2, s)


# speedup vs baseline: 1.7404x; 1.1748x over previous
import numpy as np

import jax
import jax.numpy as jnp
from jax.experimental import pallas as pl
from jax.experimental.pallas import tpu as pltpu

_TB = 512


def _moe_block_kernel(x_ref, wg_ref, bg_ref, w1_ref, b1_ref, w2_ref, b2_ref,
                      s_ref, out_ref, w1s_ref, w2s_ref):
    n_experts, _, d_hidden = w1_ref.shape

    @pl.when(pl.program_id(0) == 0)
    def _fill_weight_scratch():
        for e in range(n_experts):
            w1s_ref[:, e * d_hidden:(e + 1) * d_hidden] = (
                w1_ref[e].astype(jnp.bfloat16))
            w2s_ref[e * d_hidden:(e + 1) * d_hidden, :] = (
                w2_ref[e].astype(jnp.bfloat16))

    x = x_ref[...].astype(jnp.bfloat16)

    logits = jnp.dot(x, wg_ref[...], preferred_element_type=jnp.float32)
    logits = logits + bg_ref[...]
    expg = jnp.exp(logits - jnp.max(logits, axis=-1, keepdims=True))
    denom = jnp.sum(expg, axis=-1, keepdims=True)
    expg16 = expg.astype(jnp.bfloat16)
    gate_full = jnp.dot(expg16, s_ref[...],
                        preferred_element_type=jnp.float32)

    h = jnp.dot(x, w1s_ref[...], preferred_element_type=jnp.float32)
    hg = jnp.maximum(h + b1_ref[...], 0.0) * gate_full

    acc = jnp.dot(hg.astype(jnp.bfloat16), w2s_ref[...],
                  preferred_element_type=jnp.float32)
    acc = acc + jnp.dot(expg16, b2_ref[...],
                        preferred_element_type=jnp.float32)
    out_ref[...] = acc / denom


def kernel(features, Wg, bg, W1, b1, W2, b2):
    t, d = features.shape
    e, _, h = W1.shape
    a = W2.shape[-1]

    wgb = Wg.astype(jnp.bfloat16)
    bg2 = bg.reshape(1, e)
    b1c = b1.reshape(1, e * h)
    s = jnp.asarray(np.kron(np.eye(e, dtype=np.float32),
                            np.ones((1, h), np.float32)), dtype=jnp.bfloat16)

    grid = (t // _TB,)
    return pl.pallas_call(
        _moe_block_kernel,
        grid=grid,
        in_specs=[
            pl.BlockSpec((_TB, d), lambda i: (i, 0)),
            pl.BlockSpec((d, e), lambda i: (0, 0)),
            pl.BlockSpec((1, e), lambda i: (0, 0)),
            pl.BlockSpec((e, d, h), lambda i: (0, 0, 0)),
            pl.BlockSpec((1, e * h), lambda i: (0, 0)),
            pl.BlockSpec((e, h, a), lambda i: (0, 0, 0)),
            pl.BlockSpec((e, a), lambda i: (0, 0)),
            pl.BlockSpec((e, e * h), lambda i: (0, 0)),
        ],
        out_specs=pl.BlockSpec((_TB, a), lambda i: (i, 0)),
        out_shape=jax.ShapeDtypeStruct((t, a), jnp.float32),
        scratch_shapes=[
            pltpu.VMEM((d, e * h), jnp.bfloat16),
            pltpu.VMEM((e * h, a), jnp.bfloat16),
        ],
    )(features, wgb, bg2, W1, b1c, W2, b2, s)


# speedup vs baseline: 1.7907x; 1.0289x over previous
import numpy as np

import jax
import jax.numpy as jnp
from jax.experimental import pallas as pl
from jax.experimental.pallas import tpu as pltpu

_TB = 1024


def _moe_block_kernel(x_ref, wg_ref, bg_ref, w1_ref, b1_ref, w2_ref, b2_ref,
                      s_ref, out_ref, w1s_ref, w2s_ref):
    n_experts, _, d_hidden = w1_ref.shape

    @pl.when(pl.program_id(0) == 0)
    def _fill_weight_scratch():
        for e in range(n_experts):
            w1s_ref[:, e * d_hidden:(e + 1) * d_hidden] = (
                w1_ref[e].astype(jnp.bfloat16))
            w2s_ref[e * d_hidden:(e + 1) * d_hidden, :] = (
                w2_ref[e].astype(jnp.bfloat16))

    x = x_ref[...].astype(jnp.bfloat16)

    logits = jnp.dot(x, wg_ref[...], preferred_element_type=jnp.float32)
    logits = logits + bg_ref[...]
    expg = jnp.exp(logits - jnp.max(logits, axis=-1, keepdims=True))
    denom = jnp.sum(expg, axis=-1, keepdims=True)
    expg16 = expg.astype(jnp.bfloat16)
    gate_full = jnp.dot(expg16, s_ref[...],
                        preferred_element_type=jnp.float32)

    h = jnp.dot(x, w1s_ref[...], preferred_element_type=jnp.float32)
    hg = jnp.maximum(h + b1_ref[...], 0.0) * gate_full

    acc = jnp.dot(hg.astype(jnp.bfloat16), w2s_ref[...],
                  preferred_element_type=jnp.float32)
    acc = acc + jnp.dot(expg16, b2_ref[...],
                        preferred_element_type=jnp.float32)
    out_ref[...] = acc / denom


def kernel(features, Wg, bg, W1, b1, W2, b2):
    t, d = features.shape
    e, _, h = W1.shape
    a = W2.shape[-1]

    wgb = Wg.astype(jnp.bfloat16)
    bg2 = bg.reshape(1, e)
    b1c = b1.reshape(1, e * h)
    s = jnp.asarray(np.kron(np.eye(e, dtype=np.float32),
                            np.ones((1, h), np.float32)), dtype=jnp.bfloat16)

    grid = (t // _TB,)
    return pl.pallas_call(
        _moe_block_kernel,
        grid=grid,
        in_specs=[
            pl.BlockSpec((_TB, d), lambda i: (i, 0)),
            pl.BlockSpec((d, e), lambda i: (0, 0)),
            pl.BlockSpec((1, e), lambda i: (0, 0)),
            pl.BlockSpec((e, d, h), lambda i: (0, 0, 0)),
            pl.BlockSpec((1, e * h), lambda i: (0, 0)),
            pl.BlockSpec((e, h, a), lambda i: (0, 0, 0)),
            pl.BlockSpec((e, a), lambda i: (0, 0)),
            pl.BlockSpec((e, e * h), lambda i: (0, 0)),
        ],
        out_specs=pl.BlockSpec((_TB, a), lambda i: (i, 0)),
        out_shape=jax.ShapeDtypeStruct((t, a), jnp.float32),
        scratch_shapes=[
            pltpu.VMEM((d, e * h), jnp.bfloat16),
            pltpu.VMEM((e * h, a), jnp.bfloat16),
        ],
    )(features, wgb, bg2, W1, b1c, W2, b2, s)
